# baseline (speedup 1.0000x reference)
import functools

import jax
import jax.numpy as jnp
from jax import lax
from jax.experimental import pallas as pl
from jax.experimental.pallas import tpu as pltpu
from jax.experimental.pallas import tpu_sc as plsc

N_BATCH = 16
N_ATOMS = 8192
NBH = 64
NW = 32
L = 16
BWORDS = N_ATOMS * NBH
TJ_PER_W = 4
TA_PER_CHUNK = 16
CHUNK = TA_PER_CHUNK * 8 * 128
N_CHUNKS = TJ_PER_W * (N_ATOMS // 128) // TA_PER_CHUNK


def _safe_dist(sq):
    safe = jnp.maximum(sq, 1e-30)
    i = plsc.bitcast(safe, jnp.int32)
    y = plsc.bitcast(jnp.int32(0x5F3759DF) - (i >> 1), jnp.float32)
    xh = 0.5 * safe
    y = y * (1.5 - xh * y * y)
    y = y * (1.5 - xh * y * y)
    return sq * y


def _distances_sc(pos, nbr):
    mesh = plsc.VectorSubcoreMesh(core_axis_name="c", subcore_axis_name="s")

    @functools.partial(
        pl.kernel,
        out_type=jax.ShapeDtypeStruct((N_BATCH * N_ATOMS * NBH,), jnp.float32),
        mesh=mesh,
        scratch_types=[
            [pltpu.VMEM((64, 128), jnp.float32) for _ in range(3)],
            pltpu.VMEM((N_ATOMS,), jnp.float32),
            pltpu.VMEM((N_ATOMS,), jnp.float32),
            pltpu.VMEM((N_ATOMS,), jnp.float32),
            [pltpu.VMEM((CHUNK,), jnp.int32) for _ in range(2)],
            [pltpu.VMEM((CHUNK,), jnp.float32) for _ in range(2)],
            [pltpu.SemaphoreType.DMA for _ in range(2)],
            [pltpu.SemaphoreType.DMA for _ in range(2)],
        ],
        compiler_params=pltpu.CompilerParams(needs_layout_passes=False),
    )
    def k(pos_h, nbr_h, out_h, p2_v, x_v, y_v, z_v, idx_v, o_v, in_sem,
          out_sem):
        cid = lax.axis_index("c")
        sid = lax.axis_index("s")
        wid = cid * 16 + sid
        b = wid // 2
        half = wid % 2
        region = b * BWORDS + half * (TJ_PER_W * N_ATOMS * 8)

        for s in range(2):
            pltpu.async_copy(
                nbr_h.at[pl.ds(region + s * CHUNK, CHUNK)], idx_v[s],
                in_sem[s])
        tb = b // 8
        bi = b % 8
        for comp in range(3):
            pltpu.sync_copy(pos_h.at[comp, tb, :, bi, :], p2_v[comp])

        tabs = (x_v, y_v, z_v)

        @plsc.parallel_loop(0, N_ATOMS // L, unroll=4)
        def split_body(t):
            row = t >> 3
            col = (t & 7) * L
            for comp in range(3):
                tabs[comp][pl.ds(t * L, L)] = p2_v[comp][row, pl.ds(col, L)]

        def pair_body(g, carry):
            for s in range(2):
                c = 2 * g + s
                base = region + c * CHUNK
                a0 = (c % 4) * (TA_PER_CHUNK * 128)
                idx_c = idx_v[s]
                o_c = o_v[s]
                pltpu.make_async_copy(
                    nbr_h.at[pl.ds(base, CHUNK)], idx_c, in_sem[s]).wait()

                @pl.when(c >= 2)
                def _():
                    pltpu.make_async_copy(
                        o_c, out_h.at[pl.ds(base, CHUNK)],
                        out_sem[s]).wait()

                @plsc.parallel_loop(0, TA_PER_CHUNK * 8, unroll=4)
                def vec_body(v):
                    ta = v >> 3
                    arb = v & 7
                    a = a0 + ta * 128 + arb * L
                    cx = x_v[pl.ds(a, L)]
                    cy = y_v[pl.ds(a, L)]
                    cz = z_v[pl.ds(a, L)]
                    for tji in range(8):
                        off = ta * 1024 + tji * 128 + arb * L
                        nb = idx_c[pl.ds(off, L)]
                        dx = plsc.load_gather(x_v, [nb]) - cx
                        dy = plsc.load_gather(y_v, [nb]) - cy
                        dz = plsc.load_gather(z_v, [nb]) - cz
                        sq = dx * dx + dy * dy + dz * dz
                        o_c[pl.ds(off, L)] = _safe_dist(sq)

                pltpu.async_copy(o_c, out_h.at[pl.ds(base, CHUNK)],
                                 out_sem[s])

                @pl.when(c + 2 < N_CHUNKS)
                def _():
                    pltpu.async_copy(
                        nbr_h.at[pl.ds(base + 2 * CHUNK, CHUNK)], idx_c,
                        in_sem[s])

            return carry

        lax.fori_loop(0, N_CHUNKS // 2, pair_body, 0)
        for s in range(2):
            pltpu.make_async_copy(
                o_v[s], out_h.at[pl.ds(region, CHUNK)], out_sem[s]).wait()

    return k(pos, nbr)


def kernel(positions, neighbors):
    pos = (positions.transpose(2, 0, 1)
           .reshape(3, 2, 8, 64, 128)
           .transpose(0, 1, 3, 2, 4))
    nbr = (neighbors.astype(jnp.int32)
           .reshape(N_BATCH, 64, 128, 8, 8)
           .transpose(0, 3, 1, 4, 2)
           .reshape(-1))
    out = _distances_sc(pos, nbr)
    return (out.reshape(N_BATCH, 8, 64, 8, 128)
            .transpose(0, 2, 4, 1, 3)
            .reshape(N_BATCH, N_ATOMS, NBH))


# speedup vs baseline: 1.2358x; 1.2358x over previous
import functools

import jax
import jax.numpy as jnp
from jax import lax
from jax.experimental import pallas as pl
from jax.experimental.pallas import tpu as pltpu
from jax.experimental.pallas import tpu_sc as plsc

N_BATCH = 16
N_ATOMS = 8192
NBH = 64
NW = 32
L = 16
BWORDS = N_ATOMS * NBH
TJ_PER_W = 4
TA_PER_CHUNK = 16
CHUNK = TA_PER_CHUNK * 8 * 128
N_CHUNKS = TJ_PER_W * (N_ATOMS // 128) // TA_PER_CHUNK


def _safe_dist(sq):
    safe = jnp.maximum(sq, 1e-30)
    i = plsc.bitcast(safe, jnp.int32)
    y = plsc.bitcast(jnp.int32(0x5F3759DF) - (i >> 1), jnp.float32)
    xh = 0.5 * safe
    y = y * (1.5 - xh * y * y)
    return sq * y


def _distances_sc(pos, nbr):
    mesh = plsc.VectorSubcoreMesh(core_axis_name="c", subcore_axis_name="s")

    @functools.partial(
        pl.kernel,
        out_type=jax.ShapeDtypeStruct((N_BATCH * N_ATOMS * NBH,), jnp.float32),
        mesh=mesh,
        scratch_types=[
            [pltpu.VMEM((64, 128), jnp.float32) for _ in range(3)],
            pltpu.VMEM((N_ATOMS,), jnp.float32),
            pltpu.VMEM((N_ATOMS,), jnp.float32),
            pltpu.VMEM((N_ATOMS,), jnp.float32),
            [pltpu.VMEM((CHUNK,), jnp.int32) for _ in range(2)],
            [pltpu.VMEM((CHUNK,), jnp.float32) for _ in range(2)],
            [pltpu.SemaphoreType.DMA for _ in range(2)],
            [pltpu.SemaphoreType.DMA for _ in range(2)],
        ],
        compiler_params=pltpu.CompilerParams(needs_layout_passes=False),
    )
    def k(pos_h, nbr_h, out_h, p2_v, x_v, y_v, z_v, idx_v, o_v, in_sem,
          out_sem):
        cid = lax.axis_index("c")
        sid = lax.axis_index("s")
        wid = cid * 16 + sid
        b = wid // 2
        half = wid % 2
        region = b * BWORDS + half * (TJ_PER_W * N_ATOMS * 8)

        for s in range(2):
            pltpu.async_copy(
                nbr_h.at[pl.ds(region + s * CHUNK, CHUNK)], idx_v[s],
                in_sem[s])
        tb = b // 8
        bi = b % 8
        for comp in range(3):
            pltpu.sync_copy(pos_h.at[comp, tb, :, bi, :], p2_v[comp])

        tabs = (x_v, y_v, z_v)

        @plsc.parallel_loop(0, N_ATOMS // L, unroll=4)
        def split_body(t):
            row = t >> 3
            col = (t & 7) * L
            for comp in range(3):
                tabs[comp][pl.ds(t * L, L)] = p2_v[comp][row, pl.ds(col, L)]

        def pair_body(g, carry):
            for s in range(2):
                c = 2 * g + s
                base = region + c * CHUNK
                a0 = (c % 4) * (TA_PER_CHUNK * 128)
                idx_c = idx_v[s]
                o_c = o_v[s]
                pltpu.make_async_copy(
                    nbr_h.at[pl.ds(base, CHUNK)], idx_c, in_sem[s]).wait()

                @pl.when(c >= 2)
                def _():
                    pltpu.make_async_copy(
                        o_c, out_h.at[pl.ds(base, CHUNK)],
                        out_sem[s]).wait()

                @plsc.parallel_loop(0, TA_PER_CHUNK * 8, unroll=2)
                def vec_body(v):
                    ta = v >> 3
                    arb = v & 7
                    a = a0 + ta * 128 + arb * L
                    cx = x_v[pl.ds(a, L)]
                    cy = y_v[pl.ds(a, L)]
                    cz = z_v[pl.ds(a, L)]
                    for tji in range(8):
                        off = ta * 1024 + tji * 128 + arb * L
                        nb = idx_c[pl.ds(off, L)]
                        dx = plsc.load_gather(x_v, [nb]) - cx
                        dy = plsc.load_gather(y_v, [nb]) - cy
                        dz = plsc.load_gather(z_v, [nb]) - cz
                        sq = dx * dx + dy * dy + dz * dz
                        o_c[pl.ds(off, L)] = _safe_dist(sq)

                pltpu.async_copy(o_c, out_h.at[pl.ds(base, CHUNK)],
                                 out_sem[s])

                @pl.when(c + 2 < N_CHUNKS)
                def _():
                    pltpu.async_copy(
                        nbr_h.at[pl.ds(base + 2 * CHUNK, CHUNK)], idx_c,
                        in_sem[s])

            return carry

        lax.fori_loop(0, N_CHUNKS // 2, pair_body, 0)
        for s in range(2):
            pltpu.make_async_copy(
                o_v[s], out_h.at[pl.ds(region, CHUNK)], out_sem[s]).wait()

    return k(pos, nbr)


def kernel(positions, neighbors):
    pos = (positions.transpose(2, 0, 1)
           .reshape(3, 2, 8, 64, 128)
           .transpose(0, 1, 3, 2, 4))
    nbr = (neighbors.astype(jnp.int32)
           .reshape(N_BATCH, 64, 128, 8, 8)
           .transpose(0, 3, 1, 4, 2)
           .reshape(-1))
    out = _distances_sc(pos, nbr)
    return (out.reshape(N_BATCH, 8, 64, 8, 128)
            .transpose(0, 2, 4, 1, 3)
            .reshape(N_BATCH, N_ATOMS, NBH))


# speedup vs baseline: 1.2730x; 1.0301x over previous
import functools

import jax
import jax.numpy as jnp
from jax import lax
from jax.experimental import pallas as pl
from jax.experimental.pallas import tpu as pltpu
from jax.experimental.pallas import tpu_sc as plsc

N_BATCH = 16
N_ATOMS = 8192
NBH = 64
NW = 32
L = 16
BWORDS = N_ATOMS * NBH
TJ_PER_W = 4
TA_PER_CHUNK = 16
CHUNK = TA_PER_CHUNK * 8 * 128
N_CHUNKS = TJ_PER_W * (N_ATOMS // 128) // TA_PER_CHUNK


def _safe_dist(sq):
    i = plsc.bitcast(sq, jnp.int32)
    y = plsc.bitcast(jnp.int32(0x5F3759DF) - (i >> 1), jnp.float32)
    xh = 0.5 * sq
    y = y * (1.5 - xh * y * y)
    return sq * y


def _distances_sc(pos, nbr):
    mesh = plsc.VectorSubcoreMesh(core_axis_name="c", subcore_axis_name="s")

    @functools.partial(
        pl.kernel,
        out_type=jax.ShapeDtypeStruct((N_BATCH * N_ATOMS * NBH,), jnp.float32),
        mesh=mesh,
        scratch_types=[
            [pltpu.VMEM((64, 128), jnp.float32) for _ in range(3)],
            pltpu.VMEM((N_ATOMS,), jnp.float32),
            pltpu.VMEM((N_ATOMS,), jnp.float32),
            pltpu.VMEM((N_ATOMS,), jnp.float32),
            [pltpu.VMEM((CHUNK,), jnp.int32) for _ in range(2)],
            [pltpu.VMEM((CHUNK,), jnp.float32) for _ in range(2)],
            [pltpu.SemaphoreType.DMA for _ in range(2)],
            [pltpu.SemaphoreType.DMA for _ in range(2)],
        ],
        compiler_params=pltpu.CompilerParams(needs_layout_passes=False),
    )
    def k(pos_h, nbr_h, out_h, p2_v, x_v, y_v, z_v, idx_v, o_v, in_sem,
          out_sem):
        cid = lax.axis_index("c")
        sid = lax.axis_index("s")
        wid = cid * 16 + sid
        b = wid // 2
        half = wid % 2
        region = b * BWORDS + half * (TJ_PER_W * N_ATOMS * 8)

        for s in range(2):
            pltpu.async_copy(
                nbr_h.at[pl.ds(region + s * CHUNK, CHUNK)], idx_v[s],
                in_sem[s])
        tb = b // 8
        bi = b % 8
        for comp in range(3):
            pltpu.sync_copy(pos_h.at[comp, tb, :, bi, :], p2_v[comp])

        tabs = (x_v, y_v, z_v)

        @plsc.parallel_loop(0, N_ATOMS // L, unroll=4)
        def split_body(t):
            row = t >> 3
            col = (t & 7) * L
            for comp in range(3):
                tabs[comp][pl.ds(t * L, L)] = p2_v[comp][row, pl.ds(col, L)]

        def pair_body(g, carry):
            for s in range(2):
                c = 2 * g + s
                base = region + c * CHUNK
                a0 = (c % 4) * (TA_PER_CHUNK * 128)
                idx_c = idx_v[s]
                o_c = o_v[s]
                pltpu.make_async_copy(
                    nbr_h.at[pl.ds(base, CHUNK)], idx_c, in_sem[s]).wait()

                @pl.when(c >= 2)
                def _():
                    pltpu.make_async_copy(
                        o_c, out_h.at[pl.ds(base, CHUNK)],
                        out_sem[s]).wait()

                @plsc.parallel_loop(0, TA_PER_CHUNK * 8, unroll=2)
                def vec_body(v):
                    ta = v >> 3
                    arb = v & 7
                    a = a0 + ta * 128 + arb * L
                    cx = x_v[pl.ds(a, L)]
                    cy = y_v[pl.ds(a, L)]
                    cz = z_v[pl.ds(a, L)]
                    for tji in range(8):
                        off = ta * 1024 + tji * 128 + arb * L
                        nb = idx_c[pl.ds(off, L)]
                        dx = plsc.load_gather(x_v, [nb]) - cx
                        dy = plsc.load_gather(y_v, [nb]) - cy
                        dz = plsc.load_gather(z_v, [nb]) - cz
                        sq = dx * dx + dy * dy + dz * dz
                        o_c[pl.ds(off, L)] = _safe_dist(sq)

                pltpu.async_copy(o_c, out_h.at[pl.ds(base, CHUNK)],
                                 out_sem[s])

                @pl.when(c + 2 < N_CHUNKS)
                def _():
                    pltpu.async_copy(
                        nbr_h.at[pl.ds(base + 2 * CHUNK, CHUNK)], idx_c,
                        in_sem[s])

            return carry

        lax.fori_loop(0, N_CHUNKS // 2, pair_body, 0)
        for s in range(2):
            pltpu.make_async_copy(
                o_v[s], out_h.at[pl.ds(region, CHUNK)], out_sem[s]).wait()

    return k(pos, nbr)


def kernel(positions, neighbors):
    pos = (positions.transpose(2, 0, 1)
           .reshape(3, 2, 8, 64, 128)
           .transpose(0, 1, 3, 2, 4))
    nbr = (neighbors.astype(jnp.int32)
           .reshape(N_BATCH, 64, 128, 8, 8)
           .transpose(0, 3, 1, 4, 2)
           .reshape(-1))
    out = _distances_sc(pos, nbr)
    return (out.reshape(N_BATCH, 8, 64, 8, 128)
            .transpose(0, 2, 4, 1, 3)
            .reshape(N_BATCH, N_ATOMS, NBH))
